# Initial kernel scaffold; baseline (speedup 1.0000x reference)
import jax
import jax.numpy as jnp
from jax.experimental import pallas as pl


def kernel(inputs, targets):
    raise NotImplementedError("write your pallas kernel here")


# speedup vs baseline: 52.6799x; 52.6799x over previous
import functools

import jax
import jax.numpy as jnp
from jax.experimental import pallas as pl
from jax.experimental.pallas import tpu as pltpu

_EPS = 0.1
_K = 39935


def _sortable(b):
    return b ^ jnp.where(b < 0, jnp.int32(0x7FFFFFFF), jnp.int32(0))


def _body(tgt_ref, x_ref, out_ref, y_ref, *, rows, vocab, batch):
    i = pl.program_id(0)
    x = x_ref[...]

    b = jax.lax.bitcast_convert_type(x, jnp.int32)
    y = _sortable(b)
    y_ref[...] = y

    m = jnp.max(x, axis=1, keepdims=True)
    se = jnp.sum(jnp.exp(x - m), axis=1, keepdims=True)
    lse = m + jnp.log(se)

    lo = jnp.min(y, axis=1, keepdims=True)
    hi = jnp.max(y, axis=1, keepdims=True)

    def step(_, carry):
        lo, hi = carry
        mid = (lo >> 1) + (hi >> 1) + (lo & hi & 1) + ((lo ^ hi) & 1)
        cge = jnp.sum((y_ref[...] >= mid).astype(jnp.int32), axis=1,
                      keepdims=True)
        ge = cge >= _K
        lo = jnp.where(ge, mid, lo)
        hi = jnp.where(ge, hi, mid - 1)
        return lo, hi

    lo, hi = jax.lax.fori_loop(0, 32, step, (lo, hi), unroll=False)
    t_sort = lo
    t_val = jax.lax.bitcast_convert_type(_sortable(t_sort), jnp.float32)

    yv = y_ref[...]
    gt = yv > t_sort
    cnt_gt = jnp.sum(gt.astype(jnp.int32), axis=1, keepdims=True)
    s_gt = jnp.sum(jnp.where(gt, x, 0.0), axis=1, keepdims=True)

    tcol = jnp.stack([tgt_ref[i * rows + r] for r in range(rows)]).reshape(
        rows, 1)
    lane = jax.lax.broadcasted_iota(jnp.int32, (rows, vocab), 1)
    tmask = lane == tcol
    x_t = jnp.sum(jnp.where(tmask, x, 0.0), axis=1, keepdims=True)
    y_t = jnp.sum(jnp.where(tmask, yv, 0), axis=1, keepdims=True)
    cnt_eq_lt = jnp.sum(((yv == t_sort) & (lane < tcol)).astype(jnp.int32),
                        axis=1, keepdims=True)
    incl = (y_t > t_sort) | ((y_t == t_sort) & (cnt_gt + cnt_eq_lt < _K))

    s_topk = s_gt + (_K - cnt_gt).astype(jnp.float32) * t_val
    l_sum_topk = _K * lse - s_topk
    l_t = lse - x_t
    loss_rows = (_EPS / _K) * (l_sum_topk - jnp.where(incl, l_t, 0.0)) \
        + (1.0 - _EPS) * l_t

    part = jnp.sum(loss_rows) / batch

    @pl.when(i == 0)
    def _():
        out_ref[0, 0] = 0.0

    out_ref[0, 0] += part


def kernel(inputs, targets):
    B, V = inputs.shape
    rows = 8
    t32 = targets.astype(jnp.int32)
    out = pl.pallas_call(
        functools.partial(_body, rows=rows, vocab=V, batch=float(B)),
        grid_spec=pltpu.PrefetchScalarGridSpec(
            num_scalar_prefetch=1,
            grid=(B // rows,),
            in_specs=[pl.BlockSpec((rows, V), lambda i, t: (i, 0))],
            out_specs=pl.BlockSpec(memory_space=pltpu.SMEM),
            scratch_shapes=[pltpu.VMEM((rows, V), jnp.int32)],
        ),
        out_shape=jax.ShapeDtypeStruct((1, 1), jnp.float32),
    )(t32, inputs)
    return out[0, 0]
